# baseline (speedup 1.0000x reference)
import functools

import jax
import jax.numpy as jnp
from jax import lax
from jax.experimental import pallas as pl
from jax.experimental.pallas import tpu as pltpu
from jax.experimental.pallas import tpu_sc as plsc

_C = 8
_N = 5000
_NP = 5120
_SUB = 8
_LANE = 640
_K = 300
_KP = 320
_THRESH = 0.35
_IOU = 0.8
_BIG = 2 ** 30
_NW = 32
_RPW = _KP // _NW


def _tc_kernel(xs_ref, ys_ref, cls_ref, conf_ref,
               out_s_ref, out_l_ref, out_a_ref, sel_i_ref, sel_e_ref,
               slab_ref, s2_ref):
    f32 = jnp.float32

    for c in range(_C):
        xv = xs_ref[c, 0]
        Xv = xv
        yv = ys_ref[c, 0]
        Yv = yv
        for k in range(1, 8):
            tx = xs_ref[c, k]
            ty = ys_ref[c, k]
            xv = jnp.minimum(xv, tx)
            Xv = jnp.maximum(Xv, tx)
            yv = jnp.minimum(yv, ty)
            Yv = jnp.maximum(Yv, ty)
        slab_ref[c, 0] = xv
        slab_ref[c, 1] = yv
        slab_ref[c, 2] = Xv
        slab_ref[c, 3] = Yv
        slab_ref[c, 4] = (Xv - xv + 1.0) * (Yv - yv + 1.0)
        mf = (cls_ref[c] > _THRESH).astype(f32)
        slab_ref[c, 5] = mf
        slab_ref[c, 6] = mf * conf_ref[c]

    cls_all = cls_ref[...]
    s2_ref[...] = jnp.where(cls_all > _THRESH, cls_all, -jnp.inf)

    for t in range(_K, _KP):
        sel_i_ref[(t // _RPW) * 16 + (t % _RPW)] = -1
        sel_e_ref[(t // _RPW) * 16 + (t % _RPW)] = 0

    ci = lax.broadcasted_iota(jnp.int32, (_C, _SUB, _LANE), 0)
    si = lax.broadcasted_iota(jnp.int32, (_C, _SUB, _LANE), 1)
    li = lax.broadcasted_iota(jnp.int32, (_C, _SUB, _LANE), 2)
    flat3 = ci * _NP + si * _LANE + li

    m0 = jnp.max(s2_ref[...])

    def sel_body(t, m):
        s = s2_ref[...]
        idx = jnp.min(jnp.where(s == m, flat3, _BIG))
        s2 = jnp.where(flat3 == idx, -jnp.inf, s)
        s2_ref[...] = s2
        valid = m > -3e38
        c = idx // _NP
        i = idx - c * _NP
        l_i = i - (i // _LANE) * _LANE
        pos = (t // _RPW) * 16 + (t % _RPW)
        sel_i_ref[pos] = jnp.where(valid, idx, -1)
        sel_e_ref[pos] = l_i - (l_i // 16) * 16
        out_s_ref[t] = jnp.where(valid, m, -1.0)
        out_l_ref[t] = jnp.where(valid, c, -1)
        out_a_ref[t] = jnp.where(valid, i, -1)
        return jnp.max(s2)

    lax.fori_loop(0, _K, sel_body, m0)


@jax.jit
def _run_tc(xs4, ys4, cls3, conf3):
    return pl.pallas_call(
        _tc_kernel,
        out_shape=[
            jax.ShapeDtypeStruct((_K,), jnp.float32),
            jax.ShapeDtypeStruct((_K,), jnp.int32),
            jax.ShapeDtypeStruct((_K,), jnp.int32),
            jax.ShapeDtypeStruct((_NW * 16,), jnp.int32),
            jax.ShapeDtypeStruct((_NW * 16,), jnp.int32),
            jax.ShapeDtypeStruct((_C, 7, _SUB, _LANE), jnp.float32),
        ],
        out_specs=[
            pl.BlockSpec(memory_space=pltpu.SMEM),
            pl.BlockSpec(memory_space=pltpu.SMEM),
            pl.BlockSpec(memory_space=pltpu.SMEM),
            pl.BlockSpec(memory_space=pltpu.SMEM),
            pl.BlockSpec(memory_space=pltpu.SMEM),
            pl.BlockSpec(memory_space=pltpu.VMEM),
        ],
        scratch_shapes=[
            pltpu.VMEM((_C, _SUB, _LANE), jnp.float32),
        ],
    )(xs4, ys4, cls3, conf3)


_LANEV = None


def _lperm(v, idx):
    return lax.gather(
        v, idx[:, None],
        lax.GatherDimensionNumbers(offset_dims=(), collapsed_slice_dims=(0,),
                                   start_index_map=(0,)),
        (1,), indices_are_sorted=False, unique_indices=False,
        mode=lax.GatherScatterMode.PROMISE_IN_BOUNDS)


def _lred(v, op, lane):
    for sh in (8, 4, 2, 1):
        v = op(v, _lperm(v, lane ^ sh))
    return v


def _sc_kernel(slab_hbm, seli_hbm, sele_hbm, poses_hbm, out_hbm,
               slab_a, slab_b, bc_v, seli_v, sele_v, row_v, out_v,
               sem_a, sem_b):
    f32 = jnp.float32
    wid_ = lax.axis_index("s") * 2 + lax.axis_index("c")
    pltpu.sync_copy(seli_hbm, seli_v)
    pltpu.sync_copy(sele_hbm, sele_v)
    lane = lax.iota(jnp.int32, 16)
    negv = jnp.full((16,), -1.0, f32)
    bigv = jnp.full((16,), _BIG, jnp.int32)
    zfv = jnp.zeros((16,), f32)
    wb = pl.multiple_of(wid_ * 16, 16)
    seliw = seli_v[pl.ds(wb, 16)]
    selew = sele_v[pl.ds(wb, 16)]

    cks = [jnp.maximum(seliw[k], 0) // _NP for k in range(_RPW)]
    slabs = (slab_a, slab_b)
    sems = (sem_a, sem_b)
    cps = [pltpu.make_async_copy(slab_hbm.at[cks[k]], slabs[k % 2],
                                 sems[k % 2]) for k in range(_RPW)]
    cps[0].start()

    for k in range(_RPW):
        t = wid_ * _RPW + k
        if k + 1 < _RPW:
            cps[k + 1].start()
        cps[k].wait()
        slab_v = slabs[k % 2]
        idxrv = _lred(jnp.where(lane == k, seliw, 0), jnp.add, lane)
        validv = jnp.where(idxrv >= 0, 1.0, 0.0)
        erv = _lred(jnp.where(lane == k, selew, 0), jnp.add, lane)
        bsel = lane == erv
        idx = jnp.maximum(seliw[k], 0)
        c = cks[k]
        i = idx - c * _NP
        s_i = i // _LANE
        ib = pl.multiple_of(((i - s_i * _LANE) // 16) * 16, 16)
        x1i = _lred(jnp.where(bsel, slab_v[0, s_i, pl.ds(ib, 16)], 0.0),
                    jnp.add, lane)
        y1i = _lred(jnp.where(bsel, slab_v[1, s_i, pl.ds(ib, 16)], 0.0),
                    jnp.add, lane)
        x2i = _lred(jnp.where(bsel, slab_v[2, s_i, pl.ds(ib, 16)], 0.0),
                    jnp.add, lane)
        y2i = _lred(jnp.where(bsel, slab_v[3, s_i, pl.ds(ib, 16)], 0.0),
                    jnp.add, lane)
        areai = (x2i - x1i + 1.0) * (y2i - y1i + 1.0)

        def ch1(j, car):
            m, mi, acc, idxv = car
            s = j // (_LANE // 16)
            b = (j % (_LANE // 16)) * 16
            x1 = slab_v[0, s, pl.ds(b, 16)]
            y1 = slab_v[1, s, pl.ds(b, 16)]
            x2 = slab_v[2, s, pl.ds(b, 16)]
            y2 = slab_v[3, s, pl.ds(b, 16)]
            ar = slab_v[4, s, pl.ds(b, 16)]
            mfv = slab_v[5, s, pl.ds(b, 16)]
            cmv = slab_v[6, s, pl.ds(b, 16)]
            ww = jnp.minimum(x2, x2i) - jnp.maximum(x1, x1i) + 1.0
            hh = jnp.minimum(y2, y2i) - jnp.maximum(y1, y1i) + 1.0
            inter = ww * hh
            union = ar + areai - inter
            cond = (ww > 0.0) & (hh > 0.0) & (inter > _IOU * union)
            ind = jnp.where(cond, mfv, 0.0)
            bcv = jnp.where(cond, cmv, 0.0)
            acc = acc + ind
            bc_v[s, pl.ds(b, 16)] = bcv
            upd = bcv > m
            m = jnp.where(upd, bcv, m)
            mi = jnp.where(upd, idxv, mi)
            return m, mi, acc, idxv + 16

        m, mi, acc, _ = lax.fori_loop(0, _NP // 16, ch1,
                                      (negv, bigv, zfv, lane))
        n_ovv = _lred(acc, jnp.add, lane)
        m1 = _lred(m, jnp.maximum, lane)
        j1v = _lred(jnp.where(m == m1, mi, bigv), jnp.minimum, lane)
        w1v = jnp.where(m1 > 0.0, 1.0, 0.0)

        def rescan(e1v, e2v):
            def ch(j, car):
                m, mi, idxv = car
                s = j // (_LANE // 16)
                b = (j % (_LANE // 16)) * 16
                bcv = bc_v[s, pl.ds(b, 16)]
                bad = (idxv == e1v) | (idxv == e2v)
                bcv = jnp.where(bad, -1.0, bcv)
                upd = bcv > m
                m = jnp.where(upd, bcv, m)
                mi = jnp.where(upd, idxv, mi)
                return m, mi, idxv + 16

            m, mi, _ = lax.fori_loop(0, _NP // 16, ch, (negv, bigv, lane))
            mm = _lred(m, jnp.maximum, lane)
            jjv = _lred(jnp.where(m == mm, mi, bigv), jnp.minimum, lane)
            wv = jnp.where(mm > 0.0, 1.0, 0.0)
            return jjv, wv

        j2v, w2v = rescan(j1v, j1v)
        j3v, w3v = rescan(j1v, j2v)

        pose = zfv
        for jkv, wkv in ((j1v, w1v), (j2v, w2v), (j3v, w3v)):
            pltpu.sync_copy(poses_hbm.at[c * _NP + jkv[0]], row_v)
            pose = pose + row_v[...] * wkv

        denom = jnp.maximum(w1v + w2v + w3v, 1.0)
        gfv = validv * jnp.where(n_ovv >= 3.0, 1.0, 0.0)
        out_v[...] = (pose / denom) * gfv + (gfv - 1.0)
        pltpu.sync_copy(out_v, out_hbm.at[t])


@jax.jit
def _run_sc(slab, sel_i, sel_e, poses2):
    mesh = plsc.VectorSubcoreMesh(core_axis_name="c", subcore_axis_name="s")
    k = functools.partial(
        pl.kernel,
        out_type=jax.ShapeDtypeStruct((_KP, 16), jnp.float32),
        mesh=mesh,
        scratch_types=[
            pltpu.VMEM((7, _SUB, _LANE), jnp.float32),
            pltpu.VMEM((7, _SUB, _LANE), jnp.float32),
            pltpu.VMEM((_SUB, _LANE), jnp.float32),
            pltpu.VMEM((_NW * 16,), jnp.int32),
            pltpu.VMEM((_NW * 16,), jnp.int32),
            pltpu.VMEM((16,), jnp.float32),
            pltpu.VMEM((16,), jnp.float32),
            pltpu.SemaphoreType.DMA,
            pltpu.SemaphoreType.DMA,
        ],
    )(_sc_kernel)
    return k(slab, sel_i, sel_e, poses2)


def _prep(boxes3D, classification, poses, confidence):
    b = boxes3D.reshape(_N, _C, 8, 2)
    xs = jnp.transpose(b[..., 0], (1, 2, 0))
    ys = jnp.transpose(b[..., 1], (1, 2, 0))
    pad = ((0, 0), (0, 0), (0, _NP - _N))
    xs4 = jnp.pad(xs, pad).reshape(_C, 8, _SUB, _LANE)
    ys4 = jnp.pad(ys, pad).reshape(_C, 8, _SUB, _LANE)
    cls3 = jnp.pad(classification.reshape(_N, _C).T, ((0, 0), (0, _NP - _N)),
                   constant_values=-1.0).reshape(_C, _SUB, _LANE)
    conf3 = jnp.pad(confidence.reshape(_N, _C).T, ((0, 0), (0, _NP - _N))
                    ).reshape(_C, _SUB, _LANE)
    pos = jnp.transpose(poses.reshape(_N, _C, 12), (1, 0, 2))
    poses2 = jnp.pad(pos, ((0, 0), (0, _NP - _N), (0, 4))).reshape(_C * _NP, 16)
    return xs4, ys4, cls3, conf3, poses2


def kernel(boxes3D, classification, poses, confidence):
    xs4, ys4, cls3, conf3, poses2 = _prep(boxes3D, classification, poses,
                                          confidence)
    out_s, out_l, out_a, sel_i, sel_e, slab = _run_tc(xs4, ys4, cls3, conf3)
    pose_rows = _run_sc(slab, sel_i, sel_e, poses2)
    return out_s, out_l, pose_rows[:_K, :12], out_a


# speedup vs baseline: 1.0084x; 1.0084x over previous
import functools

import jax
import jax.numpy as jnp
from jax import lax
from jax.experimental import pallas as pl
from jax.experimental.pallas import tpu as pltpu
from jax.experimental.pallas import tpu_sc as plsc

_C = 8
_N = 5000
_NP = 5120
_SUB = 8
_LANE = 640
_K = 300
_KP = 320
_THRESH = 0.35
_IOU = 0.8
_BIG = 2 ** 30
_NW = 32
_RPW = _KP // _NW


def _tc_kernel(xs_ref, ys_ref, cls_ref, conf_ref,
               out_s_ref, out_l_ref, out_a_ref, sel_i_ref, sel_e_ref,
               slab_ref, s2_ref):
    f32 = jnp.float32

    for c in range(_C):
        xv = xs_ref[c, 0]
        Xv = xv
        yv = ys_ref[c, 0]
        Yv = yv
        for k in range(1, 8):
            tx = xs_ref[c, k]
            ty = ys_ref[c, k]
            xv = jnp.minimum(xv, tx)
            Xv = jnp.maximum(Xv, tx)
            yv = jnp.minimum(yv, ty)
            Yv = jnp.maximum(Yv, ty)
        slab_ref[c, 0] = xv
        slab_ref[c, 1] = yv
        slab_ref[c, 2] = Xv
        slab_ref[c, 3] = Yv
        slab_ref[c, 4] = (Xv - xv + 1.0) * (Yv - yv + 1.0)
        mf = (cls_ref[c] > _THRESH).astype(f32)
        slab_ref[c, 5] = mf
        slab_ref[c, 6] = mf * conf_ref[c]

    cls_all = cls_ref[...]
    s2_ref[...] = jnp.where(cls_all > _THRESH, cls_all, -jnp.inf)

    for t in range(_K, _KP):
        sel_i_ref[(t // _RPW) * 16 + (t % _RPW)] = -1
        sel_e_ref[(t // _RPW) * 16 + (t % _RPW)] = 0

    ci = lax.broadcasted_iota(jnp.int32, (_C, _SUB, _LANE), 0)
    si = lax.broadcasted_iota(jnp.int32, (_C, _SUB, _LANE), 1)
    li = lax.broadcasted_iota(jnp.int32, (_C, _SUB, _LANE), 2)
    flat3 = ci * _NP + si * _LANE + li

    def sel_body(t, _):
        s = s2_ref[...]
        m = jnp.max(s)
        idx = jnp.min(jnp.where(s == m, flat3, _BIG))
        s2_ref[...] = jnp.where(flat3 == idx, -jnp.inf, s)
        valid = m > -3e38
        c = idx // _NP
        i = idx - c * _NP
        l_i = i - (i // _LANE) * _LANE
        pos = (t // _RPW) * 16 + (t % _RPW)
        sel_i_ref[pos] = jnp.where(valid, idx, -1)
        sel_e_ref[pos] = l_i - (l_i // 16) * 16
        out_s_ref[t] = jnp.where(valid, m, -1.0)
        out_l_ref[t] = jnp.where(valid, c, -1)
        out_a_ref[t] = jnp.where(valid, i, -1)
        return 0

    lax.fori_loop(0, _K, sel_body, 0)


@jax.jit
def _run_tc(xs4, ys4, cls3, conf3):
    return pl.pallas_call(
        _tc_kernel,
        out_shape=[
            jax.ShapeDtypeStruct((_K,), jnp.float32),
            jax.ShapeDtypeStruct((_K,), jnp.int32),
            jax.ShapeDtypeStruct((_K,), jnp.int32),
            jax.ShapeDtypeStruct((_NW * 16,), jnp.int32),
            jax.ShapeDtypeStruct((_NW * 16,), jnp.int32),
            jax.ShapeDtypeStruct((_C, 7, _SUB, _LANE), jnp.float32),
        ],
        out_specs=[
            pl.BlockSpec(memory_space=pltpu.SMEM),
            pl.BlockSpec(memory_space=pltpu.SMEM),
            pl.BlockSpec(memory_space=pltpu.SMEM),
            pl.BlockSpec(memory_space=pltpu.SMEM),
            pl.BlockSpec(memory_space=pltpu.SMEM),
            pl.BlockSpec(memory_space=pltpu.VMEM),
        ],
        scratch_shapes=[
            pltpu.VMEM((_C, _SUB, _LANE), jnp.float32),
        ],
    )(xs4, ys4, cls3, conf3)


_LANEV = None


def _lperm(v, idx):
    return lax.gather(
        v, idx[:, None],
        lax.GatherDimensionNumbers(offset_dims=(), collapsed_slice_dims=(0,),
                                   start_index_map=(0,)),
        (1,), indices_are_sorted=False, unique_indices=False,
        mode=lax.GatherScatterMode.PROMISE_IN_BOUNDS)


def _lred(v, op, lane):
    for sh in (8, 4, 2, 1):
        v = op(v, _lperm(v, lane ^ sh))
    return v


def _sc_kernel(slab_hbm, seli_hbm, sele_hbm, poses_hbm, out_hbm,
               slab_a, slab_b, bc_v, seli_v, sele_v, row_v, out_v,
               sem_a, sem_b):
    f32 = jnp.float32
    wid_ = lax.axis_index("s") * 2 + lax.axis_index("c")
    pltpu.sync_copy(seli_hbm, seli_v)
    pltpu.sync_copy(sele_hbm, sele_v)
    lane = lax.iota(jnp.int32, 16)
    negv = jnp.full((16,), -1.0, f32)
    bigv = jnp.full((16,), _BIG, jnp.int32)
    zfv = jnp.zeros((16,), f32)
    wb = pl.multiple_of(wid_ * 16, 16)
    seliw = seli_v[pl.ds(wb, 16)]
    selew = sele_v[pl.ds(wb, 16)]

    cks = [jnp.maximum(seliw[k], 0) // _NP for k in range(_RPW)]
    slabs = (slab_a, slab_b)
    sems = (sem_a, sem_b)
    cps = [pltpu.make_async_copy(slab_hbm.at[cks[k]], slabs[k % 2],
                                 sems[k % 2]) for k in range(_RPW)]
    cps[0].start()

    for k in range(_RPW):
        t = wid_ * _RPW + k
        if k + 1 < _RPW:
            cps[k + 1].start()
        cps[k].wait()
        slab_v = slabs[k % 2]
        idxrv = _lred(jnp.where(lane == k, seliw, 0), jnp.add, lane)
        validv = jnp.where(idxrv >= 0, 1.0, 0.0)
        erv = _lred(jnp.where(lane == k, selew, 0), jnp.add, lane)
        bsel = lane == erv
        idx = jnp.maximum(seliw[k], 0)
        c = cks[k]
        i = idx - c * _NP
        s_i = i // _LANE
        ib = pl.multiple_of(((i - s_i * _LANE) // 16) * 16, 16)
        x1i = _lred(jnp.where(bsel, slab_v[0, s_i, pl.ds(ib, 16)], 0.0),
                    jnp.add, lane)
        y1i = _lred(jnp.where(bsel, slab_v[1, s_i, pl.ds(ib, 16)], 0.0),
                    jnp.add, lane)
        x2i = _lred(jnp.where(bsel, slab_v[2, s_i, pl.ds(ib, 16)], 0.0),
                    jnp.add, lane)
        y2i = _lred(jnp.where(bsel, slab_v[3, s_i, pl.ds(ib, 16)], 0.0),
                    jnp.add, lane)
        areai = (x2i - x1i + 1.0) * (y2i - y1i + 1.0)

        def ch1(j, car):
            m, mi, acc, idxv = car
            s = j // (_LANE // 16)
            b = (j % (_LANE // 16)) * 16
            x1 = slab_v[0, s, pl.ds(b, 16)]
            y1 = slab_v[1, s, pl.ds(b, 16)]
            x2 = slab_v[2, s, pl.ds(b, 16)]
            y2 = slab_v[3, s, pl.ds(b, 16)]
            ar = slab_v[4, s, pl.ds(b, 16)]
            mfv = slab_v[5, s, pl.ds(b, 16)]
            cmv = slab_v[6, s, pl.ds(b, 16)]
            ww = jnp.minimum(x2, x2i) - jnp.maximum(x1, x1i) + 1.0
            hh = jnp.minimum(y2, y2i) - jnp.maximum(y1, y1i) + 1.0
            inter = ww * hh
            union = ar + areai - inter
            cond = (ww > 0.0) & (hh > 0.0) & (inter > _IOU * union)
            ind = jnp.where(cond, mfv, 0.0)
            bcv = jnp.where(cond, cmv, 0.0)
            acc = acc + ind
            bc_v[s, pl.ds(b, 16)] = bcv
            upd = bcv > m
            m = jnp.where(upd, bcv, m)
            mi = jnp.where(upd, idxv, mi)
            return m, mi, acc, idxv + 16

        m, mi, acc, _ = lax.fori_loop(0, _NP // 16, ch1,
                                      (negv, bigv, zfv, lane))
        n_ovv = _lred(acc, jnp.add, lane)
        m1 = _lred(m, jnp.maximum, lane)
        j1v = _lred(jnp.where(m == m1, mi, bigv), jnp.minimum, lane)
        w1v = jnp.where(m1 > 0.0, 1.0, 0.0)

        def rescan(e1v, e2v):
            def ch(j, car):
                m, mi, idxv = car
                s = j // (_LANE // 16)
                b = (j % (_LANE // 16)) * 16
                bcv = bc_v[s, pl.ds(b, 16)]
                bad = (idxv == e1v) | (idxv == e2v)
                bcv = jnp.where(bad, -1.0, bcv)
                upd = bcv > m
                m = jnp.where(upd, bcv, m)
                mi = jnp.where(upd, idxv, mi)
                return m, mi, idxv + 16

            m, mi, _ = lax.fori_loop(0, _NP // 16, ch, (negv, bigv, lane))
            mm = _lred(m, jnp.maximum, lane)
            jjv = _lred(jnp.where(m == mm, mi, bigv), jnp.minimum, lane)
            wv = jnp.where(mm > 0.0, 1.0, 0.0)
            return jjv, wv

        j2v, w2v = rescan(j1v, j1v)
        j3v, w3v = rescan(j1v, j2v)

        pose = zfv
        for jkv, wkv in ((j1v, w1v), (j2v, w2v), (j3v, w3v)):
            pltpu.sync_copy(poses_hbm.at[c * _NP + jkv[0]], row_v)
            pose = pose + row_v[...] * wkv

        denom = jnp.maximum(w1v + w2v + w3v, 1.0)
        gfv = validv * jnp.where(n_ovv >= 3.0, 1.0, 0.0)
        out_v[...] = (pose / denom) * gfv + (gfv - 1.0)
        pltpu.sync_copy(out_v, out_hbm.at[t])


@jax.jit
def _run_sc(slab, sel_i, sel_e, poses2):
    mesh = plsc.VectorSubcoreMesh(core_axis_name="c", subcore_axis_name="s")
    k = functools.partial(
        pl.kernel,
        out_type=jax.ShapeDtypeStruct((_KP, 16), jnp.float32),
        mesh=mesh,
        scratch_types=[
            pltpu.VMEM((7, _SUB, _LANE), jnp.float32),
            pltpu.VMEM((7, _SUB, _LANE), jnp.float32),
            pltpu.VMEM((_SUB, _LANE), jnp.float32),
            pltpu.VMEM((_NW * 16,), jnp.int32),
            pltpu.VMEM((_NW * 16,), jnp.int32),
            pltpu.VMEM((16,), jnp.float32),
            pltpu.VMEM((16,), jnp.float32),
            pltpu.SemaphoreType.DMA,
            pltpu.SemaphoreType.DMA,
        ],
    )(_sc_kernel)
    return k(slab, sel_i, sel_e, poses2)


def _prep(boxes3D, classification, poses, confidence):
    b = boxes3D.reshape(_N, _C, 8, 2)
    xs = jnp.transpose(b[..., 0], (1, 2, 0))
    ys = jnp.transpose(b[..., 1], (1, 2, 0))
    pad = ((0, 0), (0, 0), (0, _NP - _N))
    xs4 = jnp.pad(xs, pad).reshape(_C, 8, _SUB, _LANE)
    ys4 = jnp.pad(ys, pad).reshape(_C, 8, _SUB, _LANE)
    cls3 = jnp.pad(classification.reshape(_N, _C).T, ((0, 0), (0, _NP - _N)),
                   constant_values=-1.0).reshape(_C, _SUB, _LANE)
    conf3 = jnp.pad(confidence.reshape(_N, _C).T, ((0, 0), (0, _NP - _N))
                    ).reshape(_C, _SUB, _LANE)
    pos = jnp.transpose(poses.reshape(_N, _C, 12), (1, 0, 2))
    poses2 = jnp.pad(pos, ((0, 0), (0, _NP - _N), (0, 4))).reshape(_C * _NP, 16)
    return xs4, ys4, cls3, conf3, poses2


def kernel(boxes3D, classification, poses, confidence):
    xs4, ys4, cls3, conf3, poses2 = _prep(boxes3D, classification, poses,
                                          confidence)
    out_s, out_l, out_a, sel_i, sel_e, slab = _run_tc(xs4, ys4, cls3, conf3)
    pose_rows = _run_sc(slab, sel_i, sel_e, poses2)
    return out_s, out_l, pose_rows[:_K, :12], out_a
